# baseline (speedup 1.0000x reference)
import functools

import jax
import jax.numpy as jnp
from jax import lax
from jax.experimental import pallas as pl
from jax.experimental.pallas import tpu as pltpu
from jax.experimental.pallas import tpu_sc as plsc

N = 16384
NNZ = 268435
D = 256
LAYERS = 3

NQ = 4
DQ = D // NQ
NS = 16
K = 128
SHARD = ((NNZ + NS * K - 1) // (NS * K)) * K
NNZ_PAD = SHARD * NS
CHUNKS = SHARD // K
ROWS_PER_TILE = N // NS


def _sc_forward(x_flat, cols2d, rows2d, vals_x):
    mesh = plsc.VectorSubcoreMesh(core_axis_name="c", subcore_axis_name="s")
    out_sds = jax.ShapeDtypeStruct((NQ * N, DQ), jnp.float32)

    @functools.partial(
        pl.kernel,
        mesh=mesh,
        out_type=[out_sds, out_sds, out_sds],
        compiler_params=pltpu.CompilerParams(use_tc_tiling_on_sc=False),
        scratch_types=[
            pltpu.VMEM((2, K), jnp.int32),
            pltpu.VMEM((2, K), jnp.int32),
            pltpu.VMEM((2, K, 16), jnp.float32),
            pltpu.VMEM((2, K, DQ), jnp.float32),
            pltpu.VMEM((256, DQ), jnp.float32),
            pltpu.VMEM_SHARED((N, DQ), jnp.float32),
            pltpu.SemaphoreType.DMA,
            pltpu.SemaphoreType.DMA((2,)),
            pltpu.SemaphoreType.DMA((2,)),
        ],
    )
    def k(x_hbm, cols_hbm, rows_hbm, vals_hbm,
          out_hbm, s0_hbm, s1_hbm,
          cidx, ridx, vv, gbuf, zbuf, acc, msem, gsem, ssem):
        c = lax.axis_index("c")
        s = lax.axis_index("s")

        z16 = jnp.zeros((16,), jnp.float32)

        def zb_body(i, _):
            for d4 in range(DQ // 16):
                zbuf[i, pl.ds(d4 * 16, 16)] = z16
            return 0

        lax.fori_loop(0, 256, zb_body, 0)

        def zero_acc():
            for j in range(ROWS_PER_TILE // 256):
                pltpu.sync_copy(
                    zbuf, acc.at[pl.ds(s * ROWS_PER_TILE + j * 256, 256)])

        zero_acc()
        plsc.subcore_barrier()

        srcs = [x_hbm, s0_hbm, s1_hbm]
        dsts = [s0_hbm, s1_hbm, out_hbm]

        for layer in range(LAYERS):
            src = srcs[layer]
            dst = dsts[layer]
            for qi in range(NQ // 2):
                q = c * 2 + qi
                crow0 = q * (NNZ_PAD // K) + s * (SHARD // K)
                rrow0 = s * (SHARD // K)

                def meta_start(i, b):
                    pltpu.async_copy(cols_hbm.at[crow0 + i], cidx.at[b], msem)
                    pltpu.async_copy(rows_hbm.at[rrow0 + i], ridx.at[b], msem)
                    pltpu.async_copy(
                        vals_hbm.at[pl.ds((rrow0 + i) * K, K)], vv.at[b], msem)

                def meta_wait(b):
                    pltpu.make_async_copy(
                        cols_hbm.at[0], cidx.at[b], msem).wait()
                    pltpu.make_async_copy(
                        rows_hbm.at[0], ridx.at[b], msem).wait()
                    pltpu.make_async_copy(
                        vals_hbm.at[pl.ds(0, K)], vv.at[b], msem).wait()

                def gather_start(b):
                    pltpu.async_copy(src.at[cidx.at[b]], gbuf.at[b],
                                     gsem.at[b])

                def gather_wait(b):
                    pltpu.make_async_copy(
                        src.at[pl.ds(0, K)], gbuf.at[b], gsem.at[b]).wait()

                def scatter_start(b):
                    pltpu.async_copy(gbuf.at[b], acc.at[ridx.at[b]],
                                     ssem.at[b], add=True)

                def scatter_wait(b):
                    pltpu.make_async_copy(
                        gbuf.at[b], acc.at[pl.ds(0, K)], ssem.at[b]).wait()

                meta_start(0, 0)
                meta_wait(0)
                gather_start(0)
                meta_start(1, 1)

                def chunk_body(i, _):
                    b = lax.rem(i, 2)
                    nb = 1 - b

                    @pl.when(i + 1 < CHUNKS)
                    def _():
                        meta_wait(nb)

                        @pl.when(i >= 1)
                        def _():
                            scatter_wait(nb)

                        gather_start(nb)

                    gather_wait(b)
                    gb = gbuf.at[b]
                    wb = vv.at[b]

                    def edge_body(t, _):
                        for u in range(4):
                            e = t * 4 + u
                            w = wb[e, :]
                            for d4 in range(DQ // 16):
                                sl = pl.ds(d4 * 16, 16)
                                gb[e, sl] = gb[e, sl] * w
                        return 0

                    lax.fori_loop(0, K // 4, edge_body, 0)
                    scatter_start(b)

                    @pl.when(i + 2 < CHUNKS)
                    def _():
                        meta_start(i + 2, b)

                    return 0

                lax.fori_loop(0, CHUNKS, chunk_body, 0)
                scatter_wait(0)
                scatter_wait(1)
                plsc.subcore_barrier()
                pltpu.sync_copy(
                    acc.at[pl.ds(s * ROWS_PER_TILE, ROWS_PER_TILE)],
                    dst.at[pl.ds(q * N + s * ROWS_PER_TILE, ROWS_PER_TILE)])
                zero_acc()
                plsc.subcore_barrier()

    return k(x_flat, cols2d, rows2d, vals_x)


def kernel(x, rows, cols, vals):
    pad = NNZ_PAD - NNZ
    spread = (jnp.arange(pad, dtype=jnp.int32) * 97) % N
    cols_p = jnp.concatenate([cols.astype(jnp.int32), spread])
    rows_p = jnp.concatenate([rows.astype(jnp.int32), spread])
    vals_p = jnp.concatenate([vals, jnp.zeros((pad,), jnp.float32)])
    vals_x = jnp.broadcast_to(vals_p[:, None], (NNZ_PAD, 16)) + 0.0
    cols4 = (cols_p[None, :]
             + (jnp.arange(NQ, dtype=jnp.int32) * N)[:, None]).reshape(-1)
    cols2d = cols4.reshape(-1, K)
    rows2d = rows_p.reshape(-1, K)
    x_flat = x.reshape(N, NQ, DQ).transpose(1, 0, 2).reshape(NQ * N, DQ)
    out_flat, _, _ = _sc_forward(x_flat, cols2d, rows2d, vals_x)
    return out_flat.reshape(NQ, N, DQ).transpose(1, 0, 2).reshape(N, D)


# speedup vs baseline: 1.1966x; 1.1966x over previous
import functools

import jax
import jax.numpy as jnp
from jax import lax
from jax.experimental import pallas as pl
from jax.experimental.pallas import tpu as pltpu
from jax.experimental.pallas import tpu_sc as plsc

N = 16384
NNZ = 268435
D = 256
LAYERS = 3

NQ = 4
DQ = D // NQ
NS = 16
K = 128
SHARD = ((NNZ + NS * K - 1) // (NS * K)) * K
NNZ_PAD = SHARD * NS
CHUNKS = SHARD // K
ROWS_PER_TILE = N // NS


def _sc_forward(x_flat, cols2d, rows2d, vals_x):
    mesh = plsc.VectorSubcoreMesh(core_axis_name="c", subcore_axis_name="s")
    out_sds = jax.ShapeDtypeStruct((NQ * N, DQ), jnp.float32)

    @functools.partial(
        pl.kernel,
        mesh=mesh,
        out_type=[out_sds, out_sds, out_sds],
        compiler_params=pltpu.CompilerParams(use_tc_tiling_on_sc=False),
        scratch_types=[
            pltpu.VMEM((8, K), jnp.int32),
            pltpu.VMEM((8, K), jnp.int32),
            pltpu.VMEM((8, K, 16), jnp.float32),
            pltpu.VMEM((3, K, DQ), jnp.float32),
            pltpu.VMEM((256, DQ), jnp.float32),
            pltpu.VMEM_SHARED((N, DQ), jnp.float32),
            pltpu.SemaphoreType.DMA((8,)),
            pltpu.SemaphoreType.DMA((3,)),
            pltpu.SemaphoreType.DMA((3,)),
        ],
    )
    def k(x_hbm, cols_hbm, rows_hbm, vals_hbm,
          out_hbm, s0_hbm, s1_hbm,
          cidx, ridx, vv, gbuf, zbuf, acc, msem, gsem, ssem):
        c = lax.axis_index("c")
        s = lax.axis_index("s")

        z16 = jnp.zeros((16,), jnp.float32)

        def zb_body(i, _):
            for d4 in range(DQ // 16):
                zbuf[i, pl.ds(d4 * 16, 16)] = z16
            return 0

        lax.fori_loop(0, 256, zb_body, 0)

        def zero_acc():
            for j in range(ROWS_PER_TILE // 256):
                pltpu.sync_copy(
                    zbuf, acc.at[pl.ds(s * ROWS_PER_TILE + j * 256, 256)])

        zero_acc()
        plsc.subcore_barrier()

        srcs = [x_hbm, s0_hbm, s1_hbm]
        dsts = [s0_hbm, s1_hbm, out_hbm]

        for layer in range(LAYERS):
            src = srcs[layer]
            dst = dsts[layer]
            for qi in range(NQ // 2):
                q = c * 2 + qi
                crow0 = q * (NNZ_PAD // K) + s * (SHARD // K)
                rrow0 = s * (SHARD // K)

                def meta_start(i):
                    m = lax.rem(i, 8)
                    pltpu.async_copy(
                        cols_hbm.at[crow0 + i], cidx.at[m], msem.at[m])
                    pltpu.async_copy(
                        rows_hbm.at[rrow0 + i], ridx.at[m], msem.at[m])
                    pltpu.async_copy(
                        vals_hbm.at[pl.ds((rrow0 + i) * K, K)], vv.at[m],
                        msem.at[m])

                def meta_wait(i):
                    m = lax.rem(i, 8)
                    pltpu.make_async_copy(
                        cols_hbm.at[0], cidx.at[m], msem.at[m]).wait()
                    pltpu.make_async_copy(
                        rows_hbm.at[0], ridx.at[m], msem.at[m]).wait()
                    pltpu.make_async_copy(
                        vals_hbm.at[pl.ds(0, K)], vv.at[m], msem.at[m]).wait()

                def gather_start(i):
                    g = lax.rem(i, 3)
                    m = lax.rem(i, 8)
                    pltpu.async_copy(src.at[cidx.at[m]], gbuf.at[g],
                                     gsem.at[g])

                def gather_wait(i):
                    g = lax.rem(i, 3)
                    pltpu.make_async_copy(
                        src.at[pl.ds(0, K)], gbuf.at[g], gsem.at[g]).wait()

                def scatter_start(i):
                    g = lax.rem(i, 3)
                    m = lax.rem(i, 8)
                    pltpu.async_copy(gbuf.at[g], acc.at[ridx.at[m]],
                                     ssem.at[g], add=True)

                def scatter_wait(i):
                    g = lax.rem(i, 3)
                    pltpu.make_async_copy(
                        gbuf.at[g], acc.at[pl.ds(0, K)], ssem.at[g]).wait()

                for j in range(4):
                    meta_start(j)
                meta_wait(0)
                meta_wait(1)
                gather_start(0)
                gather_start(1)

                def chunk_body(i, _):
                    @pl.when(i + 2 < CHUNKS)
                    def _():
                        meta_wait(i + 2)

                        @pl.when(i >= 1)
                        def _():
                            scatter_wait(i - 1)

                        gather_start(i + 2)

                    @pl.when((i >= 1) & (i + 2 >= CHUNKS))
                    def _():
                        scatter_wait(i - 1)

                    gather_wait(i)
                    g = lax.rem(i, 3)
                    m = lax.rem(i, 8)
                    gb = gbuf.at[g]
                    wb = vv.at[m]

                    def edge_body(t, _):
                        for u in range(2):
                            e = t * 2 + u
                            w = wb[e, :]
                            for d4 in range(DQ // 16):
                                sl = pl.ds(d4 * 16, 16)
                                gb[e, sl] = gb[e, sl] * w
                        return 0

                    lax.fori_loop(0, K // 2, edge_body, 0)
                    scatter_start(i)

                    @pl.when(i + 4 < CHUNKS)
                    def _():
                        meta_start(i + 4)

                    return 0

                lax.fori_loop(0, CHUNKS, chunk_body, 0)
                scatter_wait(CHUNKS - 1)
                plsc.subcore_barrier()
                pltpu.sync_copy(
                    acc.at[pl.ds(s * ROWS_PER_TILE, ROWS_PER_TILE)],
                    dst.at[pl.ds(q * N + s * ROWS_PER_TILE, ROWS_PER_TILE)])
                zero_acc()
                plsc.subcore_barrier()

    return k(x_flat, cols2d, rows2d, vals_x)


def kernel(x, rows, cols, vals):
    pad = NNZ_PAD - NNZ
    spread = (jnp.arange(pad, dtype=jnp.int32) * 97) % N
    cols_p = jnp.concatenate([cols.astype(jnp.int32), spread])
    rows_p = jnp.concatenate([rows.astype(jnp.int32), spread])
    vals_p = jnp.concatenate([vals, jnp.zeros((pad,), jnp.float32)])
    vals_x = jnp.broadcast_to(vals_p[:, None], (NNZ_PAD, 16)) + 0.0
    cols4 = (cols_p[None, :]
             + (jnp.arange(NQ, dtype=jnp.int32) * N)[:, None]).reshape(-1)
    cols2d = cols4.reshape(-1, K)
    rows2d = rows_p.reshape(-1, K)
    x_flat = x.reshape(N, NQ, DQ).transpose(1, 0, 2).reshape(NQ * N, DQ)
    out_flat, _, _ = _sc_forward(x_flat, cols2d, rows2d, vals_x)
    return out_flat.reshape(NQ, N, DQ).transpose(1, 0, 2).reshape(N, D)
